# baseline (speedup 1.0000x reference)
import jax
import jax.numpy as jnp
from jax.experimental import pallas as pl
from jax.experimental.pallas import tpu as pltpu

T = 1024
D = 768
H = 3072
G = 4
EPG = 4
E = G * EPG
KG = 2
KE = 2
CAP = 256

BH = 768
J = H // BH


def _top2_lanes(v, width):
    lane = jax.lax.broadcasted_iota(jnp.int32, (T, width), 1)
    v1 = jnp.max(v, axis=1, keepdims=True)
    i1 = jnp.min(jnp.where(v == v1, lane, width), axis=1, keepdims=True)
    v_masked = jnp.where(lane == i1, -jnp.inf, v)
    v2 = jnp.max(v_masked, axis=1, keepdims=True)
    i2 = jnp.min(jnp.where(v_masked == v2, lane, width), axis=1, keepdims=True)
    return (v1, i1), (v2, i2)


def _routing_kernel(x_ref, wgg_ref, wge_ref, route_ref, pos_ref):
    x = x_ref[:]
    gl = jnp.dot(x, wgg_ref[:], preferred_element_type=jnp.float32)
    gp = jax.nn.softmax(gl, axis=-1)
    (gv1, gi1), (gv2, gi2) = _top2_lanes(gp, G)

    el = jnp.dot(x, wge_ref[:], preferred_element_type=jnp.float32)
    ep = [jax.nn.softmax(el[:, g * EPG:(g + 1) * EPG], axis=-1) for g in range(G)]

    ws, idxs = [], []
    for gi, gv in ((gi1, gv1), (gi2, gv2)):
        sel = jnp.zeros((T, EPG), jnp.float32)
        for g in range(G):
            sel = jnp.where(gi == g, ep[g], sel)
        (ev1, ei1), (ev2, ei2) = _top2_lanes(sel, EPG)
        for ev, ei in ((ev1, ei1), (ev2, ei2)):
            ws.append(gv * ev)
            idxs.append(gi * EPG + ei)

    denom = ws[0] + ws[1] + ws[2] + ws[3] + 1e-9
    lane_e = jax.lax.broadcasted_iota(jnp.int32, (T, E), 1)
    route = jnp.zeros((T, E), jnp.float32)
    for w, fi in zip(ws, idxs):
        route = route + jnp.where(lane_e == fi, w / denom, 0.0)

    mask = (route > 0.0).astype(jnp.float32)
    ri = jax.lax.broadcasted_iota(jnp.int32, (T, T), 0)
    ci = jax.lax.broadcasted_iota(jnp.int32, (T, T), 1)
    ltri = (ri >= ci).astype(jnp.float32)
    pos = jnp.dot(ltri, mask, preferred_element_type=jnp.float32) - 1.0

    route_ref[:] = route
    pos_ref[:] = pos


def _moe_kernel(route_ref, pos_ref, x_ref, W1_ref, b1_ref, W2_ref, b2_ref,
                out_ref, xin_s, eo_s, ct_s):
    e = pl.program_id(0)
    j = pl.program_id(1)

    @pl.when(j == 0)
    def _dispatch():
        lane_e = jax.lax.broadcasted_iota(jnp.int32, (T, E), 1)
        sel = (lane_e == e)
        r = jnp.sum(jnp.where(sel, route_ref[:], 0.0), axis=1, keepdims=True)
        p = jnp.sum(jnp.where(sel, pos_ref[:], 0.0), axis=1, keepdims=True)
        c_iota = jax.lax.broadcasted_iota(jnp.int32, (T, CAP), 1)
        keep = (r > 0.0) & (p < CAP)
        PT = jnp.where((p.astype(jnp.int32) == c_iota) & keep, 1.0, 0.0)
        ct_s[:] = r * PT
        xin_s[:] = jax.lax.dot_general(
            PT, x_ref[:], (((0,), (0,)), ((), ())),
            preferred_element_type=jnp.float32)
        eo_s[:] = jnp.zeros((CAP, D), jnp.float32)

    h = jax.nn.gelu(
        jnp.dot(xin_s[:].astype(jnp.bfloat16), W1_ref[:],
                preferred_element_type=jnp.float32)
        + b1_ref[0])
    eo_s[:] += jnp.dot(h.astype(jnp.bfloat16), W2_ref[:],
                       preferred_element_type=jnp.float32)

    @pl.when(j == J - 1)
    def _combine():
        eo = eo_s[:] + b2_ref[0]
        contrib = jnp.dot(ct_s[:], eo, preferred_element_type=jnp.float32)

        @pl.when(e == 0)
        def _init():
            out_ref[:] = contrib

        @pl.when(e > 0)
        def _acc():
            out_ref[:] += contrib


def kernel(x, wg_group, wg_expert, W1, b1, W2, b2):
    wge_flat = jnp.transpose(wg_expert, (1, 0, 2)).reshape(D, G * EPG)

    route, pos = pl.pallas_call(
        _routing_kernel,
        out_shape=[jax.ShapeDtypeStruct((T, E), jnp.float32),
                   jax.ShapeDtypeStruct((T, E), jnp.float32)],
    )(x, wg_group, wge_flat)

    out = pl.pallas_call(
        _moe_kernel,
        grid=(E, J),
        in_specs=[
            pl.BlockSpec((T, E), lambda e, j: (0, 0)),
            pl.BlockSpec((T, E), lambda e, j: (0, 0)),
            pl.BlockSpec((T, D), lambda e, j: (0, 0)),
            pl.BlockSpec((D, BH), lambda e, j: (e, j)),
            pl.BlockSpec((1, 1, BH), lambda e, j: (e, 0, j)),
            pl.BlockSpec((BH, D), lambda e, j: (e * J + j, 0)),
            pl.BlockSpec((1, 1, D), lambda e, j: (e, 0, 0)),
        ],
        out_specs=pl.BlockSpec((T, D), lambda e, j: (0, 0)),
        out_shape=jax.ShapeDtypeStruct((T, D), jnp.float32),
        scratch_shapes=[pltpu.VMEM((CAP, D), jnp.float32),
                        pltpu.VMEM((CAP, D), jnp.float32),
                        pltpu.VMEM((T, CAP), jnp.float32)],
        compiler_params=pltpu.CompilerParams(
            dimension_semantics=("arbitrary", "arbitrary")),
    )(route, pos, x, W1.astype(jnp.bfloat16).reshape(E * D, H),
      b1.reshape(E, 1, H),
      W2.astype(jnp.bfloat16).reshape(E * H, D), b2.reshape(E, 1, D))
    return out


# speedup vs baseline: 2.1373x; 2.1373x over previous
import jax
import jax.numpy as jnp
from jax.experimental import pallas as pl
from jax.experimental.pallas import tpu as pltpu

T = 1024
D = 768
H = 3072
G = 4
EPG = 4
E = G * EPG
KG = 2
KE = 2
CAP = 256


def _top2_lanes(v, width):
    lane = jax.lax.broadcasted_iota(jnp.int32, (T, width), 1)
    v1 = jnp.max(v, axis=1, keepdims=True)
    i1 = jnp.min(jnp.where(v == v1, lane, width), axis=1, keepdims=True)
    v_masked = jnp.where(lane == i1, -jnp.inf, v)
    v2 = jnp.max(v_masked, axis=1, keepdims=True)
    i2 = jnp.min(jnp.where(v_masked == v2, lane, width), axis=1, keepdims=True)
    return (v1, i1), (v2, i2)


def _routing_kernel(x_ref, wgg_ref, wge_ref, route_ref, pos_ref):
    x = x_ref[:]
    gl = jnp.dot(x, wgg_ref[:], preferred_element_type=jnp.float32)
    gp = jax.nn.softmax(gl, axis=-1)
    (gv1, gi1), (gv2, gi2) = _top2_lanes(gp, G)

    el = jnp.dot(x, wge_ref[:], preferred_element_type=jnp.float32)
    ep = [jax.nn.softmax(el[:, g * EPG:(g + 1) * EPG], axis=-1) for g in range(G)]

    ws, idxs = [], []
    for gi, gv in ((gi1, gv1), (gi2, gv2)):
        sel = jnp.zeros((T, EPG), jnp.float32)
        for g in range(G):
            sel = jnp.where(gi == g, ep[g], sel)
        (ev1, ei1), (ev2, ei2) = _top2_lanes(sel, EPG)
        for ev, ei in ((ev1, ei1), (ev2, ei2)):
            ws.append(gv * ev)
            idxs.append(gi * EPG + ei)

    denom = ws[0] + ws[1] + ws[2] + ws[3] + 1e-9
    lane_e = jax.lax.broadcasted_iota(jnp.int32, (T, E), 1)
    route = jnp.zeros((T, E), jnp.float32)
    for w, fi in zip(ws, idxs):
        route = route + jnp.where(lane_e == fi, w / denom, 0.0)

    mask = (route > 0.0).astype(jnp.float32)
    ri = jax.lax.broadcasted_iota(jnp.int32, (T, T), 0)
    ci = jax.lax.broadcasted_iota(jnp.int32, (T, T), 1)
    ltri = (ri >= ci).astype(jnp.float32)
    pos = jnp.dot(ltri, mask, preferred_element_type=jnp.float32) - 1.0

    route_ref[:] = route
    pos_ref[:] = pos


def _moe_kernel(route_ref, pos_ref, x_ref, W1_ref, b1_ref, W2_ref, b2_ref,
                out_ref):
    e = pl.program_id(0)

    lane_e = jax.lax.broadcasted_iota(jnp.int32, (T, E), 1)
    sel = (lane_e == e)
    r = jnp.sum(jnp.where(sel, route_ref[:], 0.0), axis=1, keepdims=True)
    p = jnp.sum(jnp.where(sel, pos_ref[:], 0.0), axis=1, keepdims=True)
    c_iota = jax.lax.broadcasted_iota(jnp.int32, (T, CAP), 1)
    keep = (r > 0.0) & (p < CAP)
    PT = jnp.where((p.astype(jnp.int32) == c_iota) & keep, 1.0, 0.0)

    xin = jax.lax.dot_general(
        PT, x_ref[:], (((0,), (0,)), ((), ())),
        preferred_element_type=jnp.float32)
    h = jax.nn.gelu(
        jnp.dot(xin, W1_ref[:], preferred_element_type=jnp.float32)
        + b1_ref[0])
    eo = (jnp.dot(h, W2_ref[:], preferred_element_type=jnp.float32)
          + b2_ref[0])
    contrib = jnp.dot(r * PT, eo, preferred_element_type=jnp.float32)

    @pl.when(e == 0)
    def _init():
        out_ref[:] = contrib

    @pl.when(e > 0)
    def _acc():
        out_ref[:] += contrib


def kernel(x, wg_group, wg_expert, W1, b1, W2, b2):
    wge_flat = jnp.transpose(wg_expert, (1, 0, 2)).reshape(D, G * EPG)

    route, pos = pl.pallas_call(
        _routing_kernel,
        out_shape=[jax.ShapeDtypeStruct((T, E), jnp.float32),
                   jax.ShapeDtypeStruct((T, E), jnp.float32)],
    )(x, wg_group, wge_flat)

    out = pl.pallas_call(
        _moe_kernel,
        grid=(E,),
        in_specs=[
            pl.BlockSpec((T, E), lambda e: (0, 0)),
            pl.BlockSpec((T, E), lambda e: (0, 0)),
            pl.BlockSpec((T, D), lambda e: (0, 0)),
            pl.BlockSpec((D, H), lambda e: (e, 0)),
            pl.BlockSpec((1, 1, H), lambda e: (e, 0, 0)),
            pl.BlockSpec((H, D), lambda e: (e, 0)),
            pl.BlockSpec((1, 1, D), lambda e: (e, 0, 0)),
        ],
        out_specs=pl.BlockSpec((T, D), lambda e: (0, 0)),
        out_shape=jax.ShapeDtypeStruct((T, D), jnp.float32),
        compiler_params=pltpu.CompilerParams(
            dimension_semantics=("arbitrary",)),
    )(route, pos, x, W1.reshape(E * D, H), b1.reshape(E, 1, H),
      W2.reshape(E * H, D), b2.reshape(E, 1, D))
    return out


# speedup vs baseline: 2.1962x; 1.0275x over previous
import jax
import jax.numpy as jnp
from jax.experimental import pallas as pl
from jax.experimental.pallas import tpu as pltpu

T = 1024
D = 768
H = 3072
G = 4
EPG = 4
E = G * EPG
KG = 2
KE = 2
CAP = 256


def _top2_lanes(v, width):
    lane = jax.lax.broadcasted_iota(jnp.int32, (T, width), 1)
    v1 = jnp.max(v, axis=1, keepdims=True)
    i1 = jnp.min(jnp.where(v == v1, lane, width), axis=1, keepdims=True)
    v_masked = jnp.where(lane == i1, -jnp.inf, v)
    v2 = jnp.max(v_masked, axis=1, keepdims=True)
    i2 = jnp.min(jnp.where(v_masked == v2, lane, width), axis=1, keepdims=True)
    return (v1, i1), (v2, i2)


def _routing(x, wgg, wge):
    gl = jnp.dot(x, wgg, preferred_element_type=jnp.float32)
    gp = jax.nn.softmax(gl, axis=-1)
    (gv1, gi1), (gv2, gi2) = _top2_lanes(gp, G)

    el = jnp.dot(x, wge, preferred_element_type=jnp.float32)
    ep = [jax.nn.softmax(el[:, g * EPG:(g + 1) * EPG], axis=-1) for g in range(G)]

    ws, idxs = [], []
    for gi, gv in ((gi1, gv1), (gi2, gv2)):
        sel = jnp.zeros((T, EPG), jnp.float32)
        for g in range(G):
            sel = jnp.where(gi == g, ep[g], sel)
        (ev1, ei1), (ev2, ei2) = _top2_lanes(sel, EPG)
        for ev, ei in ((ev1, ei1), (ev2, ei2)):
            ws.append(gv * ev)
            idxs.append(gi * EPG + ei)

    denom = ws[0] + ws[1] + ws[2] + ws[3] + 1e-9
    lane_e = jax.lax.broadcasted_iota(jnp.int32, (T, E), 1)
    route = jnp.zeros((T, E), jnp.float32)
    for w, fi in zip(ws, idxs):
        route = route + jnp.where(lane_e == fi, w / denom, 0.0)

    mask = (route > 0.0).astype(jnp.float32)
    ri = jax.lax.broadcasted_iota(jnp.int32, (T, T), 0)
    ci = jax.lax.broadcasted_iota(jnp.int32, (T, T), 1)
    ltri = (ri >= ci).astype(jnp.float32)
    pos = jnp.dot(ltri, mask, preferred_element_type=jnp.float32) - 1.0
    return route, pos


def _moe_kernel(x_ref, wgg_ref, wge_ref, W1_ref, b1_ref, W2_ref, b2_ref,
                out_ref, route_s, pos_s):
    e = pl.program_id(0)

    @pl.when(e == 0)
    def _do_routing():
        route, pos = _routing(x_ref[:], wgg_ref[:], wge_ref[:])
        route_s[:] = route
        pos_s[:] = pos

    lane_e = jax.lax.broadcasted_iota(jnp.int32, (T, E), 1)
    sel = (lane_e == e)
    r = jnp.sum(jnp.where(sel, route_s[:], 0.0), axis=1, keepdims=True)
    p = jnp.sum(jnp.where(sel, pos_s[:], 0.0), axis=1, keepdims=True)
    c_iota = jax.lax.broadcasted_iota(jnp.int32, (T, CAP), 1)
    keep = (r > 0.0) & (p < CAP)
    PT = jnp.where((p.astype(jnp.int32) == c_iota) & keep, 1.0, 0.0)

    xin = jax.lax.dot_general(
        PT, x_ref[:], (((0,), (0,)), ((), ())),
        preferred_element_type=jnp.float32)
    h = jax.nn.gelu(
        jnp.dot(xin, W1_ref[:], preferred_element_type=jnp.float32)
        + b1_ref[0])
    eo = (jnp.dot(h, W2_ref[:], preferred_element_type=jnp.float32)
          + b2_ref[0])
    contrib = jnp.dot(r * PT, eo, preferred_element_type=jnp.float32)

    @pl.when(e == 0)
    def _init():
        out_ref[:] = contrib

    @pl.when(e > 0)
    def _acc():
        out_ref[:] += contrib


def kernel(x, wg_group, wg_expert, W1, b1, W2, b2):
    wge_flat = jnp.transpose(wg_expert, (1, 0, 2)).reshape(D, G * EPG)

    out = pl.pallas_call(
        _moe_kernel,
        grid=(E,),
        in_specs=[
            pl.BlockSpec((T, D), lambda e: (0, 0)),
            pl.BlockSpec((D, G), lambda e: (0, 0)),
            pl.BlockSpec((D, G * EPG), lambda e: (0, 0)),
            pl.BlockSpec((D, H), lambda e: (e, 0)),
            pl.BlockSpec((1, 1, H), lambda e: (e, 0, 0)),
            pl.BlockSpec((H, D), lambda e: (e, 0)),
            pl.BlockSpec((1, 1, D), lambda e: (e, 0, 0)),
        ],
        out_specs=pl.BlockSpec((T, D), lambda e: (0, 0)),
        out_shape=jax.ShapeDtypeStruct((T, D), jnp.float32),
        scratch_shapes=[pltpu.VMEM((T, E), jnp.float32),
                        pltpu.VMEM((T, E), jnp.float32)],
        compiler_params=pltpu.CompilerParams(
            dimension_semantics=("arbitrary",)),
    )(x, wg_group, wge_flat, W1.reshape(E * D, H), b1.reshape(E, 1, H),
      W2.reshape(E * H, D), b2.reshape(E, 1, D))
    return out
